# baseline (speedup 1.0000x reference)
import jax
import jax.numpy as jnp
from jax.experimental import pallas as pl
from jax.experimental.pallas import tpu as pltpu

_CHUNK = 64 * 20000


def _body(x_ref, out_ref, acc_ref):
    i = pl.program_id(0)
    nblk = pl.num_programs(0)

    @pl.when(i == 0)
    def _init():
        acc_ref[...] = jnp.zeros_like(acc_ref)

    x = x_ref[...].reshape(_CHUNK // 128, 128)
    acc_ref[...] = jnp.maximum(acc_ref[...], jnp.max(x, axis=0, keepdims=True))

    @pl.when(i == nblk - 1)
    def _final():
        out_ref[...] = jnp.broadcast_to(jnp.max(acc_ref[...]), (16, 64))


def kernel(query, memories, W_dec, b_dec):
    flat = memories.reshape(-1)
    grid = flat.shape[0] // _CHUNK

    out = pl.pallas_call(
        _body,
        grid=(grid,),
        in_specs=[pl.BlockSpec((_CHUNK,), lambda i: (i,))],
        out_specs=pl.BlockSpec((16, 64), lambda i: (0, 0)),
        out_shape=jax.ShapeDtypeStruct((16, 64), jnp.float32),
        scratch_shapes=[pltpu.VMEM((1, 128), jnp.float32)],
        compiler_params=pltpu.CompilerParams(
            dimension_semantics=("arbitrary",),
        ),
    )(flat)
    return out


# speedup vs baseline: 1.4056x; 1.4056x over previous
import jax
import jax.numpy as jnp
from jax.experimental import pallas as pl
from jax.experimental.pallas import tpu as pltpu

_DIM = 64
_Q = 16
_EPS = 1e-8


def _scan_body(q_ref, x_ref, mem_ref, w_ref, b_ref, out_ref,
               bsim_ref, bidx_ref, gbuf_ref, sem):
    i = pl.program_id(0)
    nblk = pl.num_programs(0)
    blk = x_ref.shape[0]

    @pl.when(i == 0)
    def _init():
        bsim_ref[...] = jnp.full_like(bsim_ref, -jnp.inf)
        bidx_ref[...] = jnp.zeros_like(bidx_ref)

    q = q_ref[...]
    qn = q / (jnp.sqrt(jnp.sum(q * q, axis=1, keepdims=True)) + _EPS)

    x = x_ref[...]
    s = jax.lax.dot_general(qn, x, (((1,), (1,)), ((), ())),
                            preferred_element_type=jnp.float32)
    ones = jnp.ones((1, _DIM), jnp.float32)
    t = jax.lax.dot_general(ones, x * x, (((1,), (1,)), ((), ())),
                            preferred_element_type=jnp.float32)
    sims = s * (1.0 / (jnp.sqrt(t) + _EPS))

    bmax = jnp.max(sims, axis=1, keepdims=True)
    col = jax.lax.broadcasted_iota(jnp.int32, sims.shape, 1)
    lidx = jnp.min(jnp.where(sims >= bmax, col, blk), axis=1, keepdims=True)

    upd = bmax > bsim_ref[...]
    bsim_ref[...] = jnp.where(upd, bmax, bsim_ref[...])
    bidx_ref[...] = jnp.where(upd, i * blk + lidx, bidx_ref[...])

    @pl.when(i == nblk - 1)
    def _final():
        bidx = bidx_ref[...]
        rowq = jax.lax.broadcasted_iota(jnp.int32, (_Q, 1), 0)
        cps = []
        for qi in range(_Q):
            idx = jnp.sum(jnp.where(rowq == qi, bidx, 0))
            cp = pltpu.make_async_copy(
                mem_ref.at[pl.ds(idx, 1), :], gbuf_ref.at[pl.ds(qi, 1), :], sem)
            cp.start()
            cps.append(cp)
        for cp in cps:
            cp.wait()
        bm = gbuf_ref[...]
        o = jax.lax.dot_general(bm, w_ref[...], (((1,), (1,)), ((), ())),
                                preferred_element_type=jnp.float32)
        o = o + b_ref[...]
        gate = (bsim_ref[...] > 0.6).astype(jnp.float32)
        out_ref[...] = o * gate


def kernel(query, memories, W_dec, b_dec):
    cap = memories.shape[0]
    blk = 40000
    grid = cap // blk
    b2 = b_dec.reshape(1, _DIM)

    out = pl.pallas_call(
        _scan_body,
        grid=(grid,),
        in_specs=[
            pl.BlockSpec((_Q, _DIM), lambda i: (0, 0)),
            pl.BlockSpec((blk, _DIM), lambda i: (i, 0)),
            pl.BlockSpec(memory_space=pl.ANY),
            pl.BlockSpec((_DIM, _DIM), lambda i: (0, 0)),
            pl.BlockSpec((1, _DIM), lambda i: (0, 0)),
        ],
        out_specs=pl.BlockSpec((_Q, _DIM), lambda i: (0, 0)),
        out_shape=jax.ShapeDtypeStruct((_Q, _DIM), jnp.float32),
        scratch_shapes=[
            pltpu.VMEM((_Q, 1), jnp.float32),
            pltpu.VMEM((_Q, 1), jnp.int32),
            pltpu.VMEM((_Q, _DIM), jnp.float32),
            pltpu.SemaphoreType.DMA,
        ],
        compiler_params=pltpu.CompilerParams(
            dimension_semantics=("arbitrary",),
            vmem_limit_bytes=120 * 1024 * 1024,
        ),
    )(query, memories, memories, W_dec, b2)
    return out
